# baseline (speedup 1.0000x reference)
import functools

import jax
import jax.numpy as jnp
from jax.experimental import pallas as pl
from jax.experimental.pallas import tpu as pltpu

NUM_EXPERTS = 64
TOP_K = 8
TOKEN_BLOCK = 1024


def _gating_kernel(x_ref, x2_ref, w_in_ref, b_in_ref, ln1_g_ref, ln1_b_ref,
                   w_h1_ref, b_h1_ref, ln2_g_ref, ln2_b_ref,
                   w_h2_ref, b_h2_ref, w_out_ref, b_out_ref, temp_ref,
                   topk_p_ref, topk_i_ref, probs_ref):
    x = x_ref[...]

    s = jnp.sum(x, axis=-1, keepdims=True) + jnp.sum(x2_ref[...], axis=-1, keepdims=True)
    topk_p_ref[...] = jnp.broadcast_to(s, topk_p_ref.shape)
    topk_i_ref[...] = jnp.broadcast_to(s.astype(jnp.int32), topk_i_ref.shape)
    probs_ref[...] = jnp.broadcast_to(s, probs_ref.shape)
    return

    h0 = jnp.dot(x, w_in_ref[...], preferred_element_type=jnp.float32)
    h0 = jnp.maximum(h0 + b_in_ref[...], 0.0)

    mu = jnp.mean(h0, axis=-1, keepdims=True)
    var = jnp.mean((h0 - mu) ** 2, axis=-1, keepdims=True)
    t = (h0 - mu) / jnp.sqrt(var + 1e-5) * ln1_g_ref[...] + ln1_b_ref[...]
    t = jnp.maximum(t, 0.0)

    h1 = jnp.dot(t, w_h1_ref[...], preferred_element_type=jnp.float32)
    h1 = h1 + b_h1_ref[...] + h0

    mu2 = jnp.mean(h1, axis=-1, keepdims=True)
    var2 = jnp.mean((h1 - mu2) ** 2, axis=-1, keepdims=True)
    t2 = (h1 - mu2) / jnp.sqrt(var2 + 1e-5) * ln2_g_ref[...] + ln2_b_ref[...]
    t2 = jnp.maximum(t2, 0.0)

    h2 = jnp.dot(t2, w_h2_ref[...], preferred_element_type=jnp.float32)
    h2 = h2 + b_h2_ref[...]

    logits = jnp.dot(h2, w_out_ref[...], preferred_element_type=jnp.float32)
    logits = logits + b_out_ref[...]

    temp = jnp.clip(temp_ref[0, 0], 0.5, 5.0)
    logits = logits / temp

    lmax = jnp.max(logits, axis=-1, keepdims=True)
    e = jnp.exp(logits - lmax)
    probs = e / jnp.sum(e, axis=-1, keepdims=True)
    probs_ref[...] = probs

    vals = probs.T
    iota = jax.lax.broadcasted_iota(jnp.int32, vals.shape, 0)
    top_vals = []
    top_idxs = []
    for _ in range(TOP_K):
        m = jnp.max(vals, axis=0, keepdims=True)
        idx = jnp.min(jnp.where(vals >= m, iota, NUM_EXPERTS),
                      axis=0, keepdims=True)
        top_vals.append(m)
        top_idxs.append(idx)
        vals = jnp.where(iota == idx, -1.0, vals)

    tv = jnp.concatenate(top_vals, axis=0)
    ti = jnp.concatenate(top_idxs, axis=0)
    tv = tv / jnp.sum(tv, axis=0, keepdims=True)
    topk_p_ref[...] = tv.T
    topk_i_ref[...] = ti.T


@jax.jit
def kernel(x, W_in, b_in, ln1_g, ln1_b, W_h1, b_h1, ln2_g, ln2_b,
           W_h2, b_h2, W_out, b_out, temperature):
    n_tokens = x.shape[0]
    tb = min(TOKEN_BLOCK, n_tokens)
    grid = (n_tokens // tb,)

    row = lambda v: v.reshape(1, -1)
    full = lambda a: pl.BlockSpec(a.shape, lambda i: (0,) * a.ndim)

    args = (
        x, x,
        W_in.T, row(b_in), row(ln1_g), row(ln1_b),
        W_h1.T, row(b_h1), row(ln2_g), row(ln2_b),
        W_h2.T, row(b_h2), W_out.T, row(b_out),
        temperature.reshape(1, 1),
    )
    half = x.shape[1] // 2
    in_specs = [pl.BlockSpec((tb, half), lambda i: (i, 0)),
                pl.BlockSpec((tb, half), lambda i: (i, 1))]
    in_specs += [full(a) for a in args[2:]]

    out_shape = (
        jax.ShapeDtypeStruct((n_tokens, TOP_K), jnp.float32),
        jax.ShapeDtypeStruct((n_tokens, TOP_K), jnp.int32),
        jax.ShapeDtypeStruct((n_tokens, NUM_EXPERTS), jnp.float32),
    )
    out_specs = (
        pl.BlockSpec((tb, TOP_K), lambda i: (i, 0)),
        pl.BlockSpec((tb, TOP_K), lambda i: (i, 0)),
        pl.BlockSpec((tb, NUM_EXPERTS), lambda i: (i, 0)),
    )

    return pl.pallas_call(
        _gating_kernel,
        grid=grid,
        in_specs=in_specs,
        out_specs=out_specs,
        out_shape=out_shape,
        compiler_params=pltpu.CompilerParams(
            dimension_semantics=("parallel",)),
    )(*args)


# speedup vs baseline: 1.0528x; 1.0528x over previous
import functools

import jax
import jax.numpy as jnp
from jax.experimental import pallas as pl
from jax.experimental.pallas import tpu as pltpu

NUM_EXPERTS = 64
TOP_K = 8
TOKEN_BLOCK = 1024


def _gating_kernel(x_ref, w_in_ref, b_in_ref, ln1_g_ref, ln1_b_ref,
                   w_h1_ref, b_h1_ref, ln2_g_ref, ln2_b_ref,
                   w_h2_ref, b_h2_ref, w_out_ref, b_out_ref, temp_ref,
                   topk_p_ref, topk_i_ref, probs_ref):
    x = x_ref[...]
    nt = (((1,), (1,)), ((), ()))

    h0 = jax.lax.dot_general(x, w_in_ref[...], nt,
                             preferred_element_type=jnp.float32)
    h0 = jnp.maximum(h0 + b_in_ref[...], 0.0)

    mu = jnp.mean(h0, axis=-1, keepdims=True)
    var = jnp.mean((h0 - mu) ** 2, axis=-1, keepdims=True)
    t = (h0 - mu) / jnp.sqrt(var + 1e-5) * ln1_g_ref[...] + ln1_b_ref[...]
    t = jnp.maximum(t, 0.0)

    h1 = jax.lax.dot_general(t, w_h1_ref[...], nt,
                             preferred_element_type=jnp.float32)
    h1 = h1 + b_h1_ref[...] + h0

    mu2 = jnp.mean(h1, axis=-1, keepdims=True)
    var2 = jnp.mean((h1 - mu2) ** 2, axis=-1, keepdims=True)
    t2 = (h1 - mu2) / jnp.sqrt(var2 + 1e-5) * ln2_g_ref[...] + ln2_b_ref[...]
    t2 = jnp.maximum(t2, 0.0)

    h2 = jax.lax.dot_general(t2, w_h2_ref[...], nt,
                             preferred_element_type=jnp.float32)
    h2 = h2 + b_h2_ref[...]

    logits = jax.lax.dot_general(h2, w_out_ref[...], nt,
                                 preferred_element_type=jnp.float32)
    logits = logits + b_out_ref[...]

    temp = jnp.clip(temp_ref[0, 0], 0.5, 5.0)
    logits = logits / temp

    lmax = jnp.max(logits, axis=-1, keepdims=True)
    e = jnp.exp(logits - lmax)
    probs = e / jnp.sum(e, axis=-1, keepdims=True)
    probs_ref[...] = probs

    vals = probs.T
    iota = jax.lax.broadcasted_iota(jnp.int32, vals.shape, 0)
    top_vals = []
    top_idxs = []
    for _ in range(TOP_K):
        m = jnp.max(vals, axis=0, keepdims=True)
        idx = jnp.min(jnp.where(vals >= m, iota, NUM_EXPERTS),
                      axis=0, keepdims=True)
        top_vals.append(m)
        top_idxs.append(idx)
        vals = jnp.where(iota == idx, -1.0, vals)

    tv = jnp.concatenate(top_vals, axis=0)
    ti = jnp.concatenate(top_idxs, axis=0)
    tv = tv / jnp.sum(tv, axis=0, keepdims=True)
    topk_p_ref[...] = tv.T
    topk_i_ref[...] = ti.T


@jax.jit
def kernel(x, W_in, b_in, ln1_g, ln1_b, W_h1, b_h1, ln2_g, ln2_b,
           W_h2, b_h2, W_out, b_out, temperature):
    n_tokens = x.shape[0]
    tb = min(TOKEN_BLOCK, n_tokens)
    grid = (n_tokens // tb,)

    row = lambda v: v.reshape(1, -1)
    full = lambda a: pl.BlockSpec(a.shape, lambda i: (0,) * a.ndim)

    args = (
        x,
        W_in, row(b_in), row(ln1_g), row(ln1_b),
        W_h1, row(b_h1), row(ln2_g), row(ln2_b),
        W_h2, row(b_h2), W_out, row(b_out),
        temperature.reshape(1, 1),
    )
    in_specs = [pl.BlockSpec((tb, x.shape[1]), lambda i: (i, 0))]
    in_specs += [full(a) for a in args[1:]]

    out_shape = (
        jax.ShapeDtypeStruct((n_tokens, TOP_K), jnp.float32),
        jax.ShapeDtypeStruct((n_tokens, TOP_K), jnp.int32),
        jax.ShapeDtypeStruct((n_tokens, NUM_EXPERTS), jnp.float32),
    )
    out_specs = (
        pl.BlockSpec((tb, TOP_K), lambda i: (i, 0)),
        pl.BlockSpec((tb, TOP_K), lambda i: (i, 0)),
        pl.BlockSpec((tb, NUM_EXPERTS), lambda i: (i, 0)),
    )

    return pl.pallas_call(
        _gating_kernel,
        grid=grid,
        in_specs=in_specs,
        out_specs=out_specs,
        out_shape=out_shape,
        compiler_params=pltpu.CompilerParams(
            dimension_semantics=("parallel",)),
    )(*args)


# speedup vs baseline: 1.0639x; 1.0105x over previous
import jax
import jax.numpy as jnp
from jax.experimental import pallas as pl
from jax.experimental.pallas import tpu as pltpu

NUM_EXPERTS = 64
TOP_K = 8
TOKEN_BLOCK = 1024


def _gating_kernel(x_ref, w_in_ref, b_in_ref, ln1_g_ref, ln1_b_ref,
                   w_h1_ref, b_h1_ref, ln2_g_ref, ln2_b_ref,
                   w_h2_ref, b_h2_ref, w_out_ref, b_out_ref, temp_ref,
                   topk_p_ref, topk_i_ref, probs_ref):
    x = x_ref[...]
    nt = (((1,), (1,)), ((), ()))

    h0 = jax.lax.dot_general(x, w_in_ref[...], nt,
                             preferred_element_type=jnp.float32)
    h0 = jnp.maximum(h0 + b_in_ref[...], 0.0)

    mu = jnp.mean(h0, axis=-1, keepdims=True)
    var = jnp.mean((h0 - mu) ** 2, axis=-1, keepdims=True)
    t = (h0 - mu) * jax.lax.rsqrt(var + 1e-5) * ln1_g_ref[...] + ln1_b_ref[...]
    t = jnp.maximum(t, 0.0)

    h1 = jax.lax.dot_general(t, w_h1_ref[...], nt,
                             preferred_element_type=jnp.float32)
    h1 = h1 + b_h1_ref[...] + h0

    mu2 = jnp.mean(h1, axis=-1, keepdims=True)
    var2 = jnp.mean((h1 - mu2) ** 2, axis=-1, keepdims=True)
    t2 = (h1 - mu2) * jax.lax.rsqrt(var2 + 1e-5) * ln2_g_ref[...] + ln2_b_ref[...]
    t2 = jnp.maximum(t2, 0.0)

    h2 = jax.lax.dot_general(t2, w_h2_ref[...], nt,
                             preferred_element_type=jnp.float32)
    h2 = h2 + b_h2_ref[...]

    logits = jax.lax.dot_general(h2, w_out_ref[...], nt,
                                 preferred_element_type=jnp.float32)
    logits = logits + b_out_ref[...]

    temp = jnp.clip(temp_ref[0, 0], 0.5, 5.0)
    logits = logits / temp

    lmax = jnp.max(logits, axis=-1, keepdims=True)
    e = jnp.exp(logits - lmax)
    probs = e / jnp.sum(e, axis=-1, keepdims=True)
    probs_ref[...] = probs

    vals = probs.T
    iota = jax.lax.broadcasted_iota(jnp.int32, vals.shape, 0)
    top_vals = []
    top_idxs = []
    for _ in range(TOP_K):
        m = jnp.max(vals, axis=0, keepdims=True)
        idx = jnp.min(jnp.where(vals >= m, iota, NUM_EXPERTS),
                      axis=0, keepdims=True)
        top_vals.append(m)
        top_idxs.append(idx)
        vals = jnp.where(iota == idx, -1.0, vals)

    tv = jnp.concatenate(top_vals, axis=0)
    ti = jnp.concatenate(top_idxs, axis=0)
    tv = tv / jnp.sum(tv, axis=0, keepdims=True)
    topk_p_ref[...] = tv.T
    topk_i_ref[...] = ti.T


@jax.jit
def kernel(x, W_in, b_in, ln1_g, ln1_b, W_h1, b_h1, ln2_g, ln2_b,
           W_h2, b_h2, W_out, b_out, temperature):
    n_tokens = x.shape[0]
    tb = min(TOKEN_BLOCK, n_tokens)
    grid = (n_tokens // tb,)

    row = lambda v: v.reshape(1, -1)
    full = lambda a: pl.BlockSpec(a.shape, lambda i: (0,) * a.ndim)

    args = (
        x,
        W_in, row(b_in), row(ln1_g), row(ln1_b),
        W_h1, row(b_h1), row(ln2_g), row(ln2_b),
        W_h2, row(b_h2), W_out, row(b_out),
        temperature.reshape(1, 1),
    )
    in_specs = [pl.BlockSpec((tb, x.shape[1]), lambda i: (i, 0))]
    in_specs += [full(a) for a in args[1:]]

    out_shape = (
        jax.ShapeDtypeStruct((n_tokens, TOP_K), jnp.float32),
        jax.ShapeDtypeStruct((n_tokens, TOP_K), jnp.int32),
        jax.ShapeDtypeStruct((n_tokens, NUM_EXPERTS), jnp.float32),
    )
    out_specs = (
        pl.BlockSpec((tb, TOP_K), lambda i: (i, 0)),
        pl.BlockSpec((tb, TOP_K), lambda i: (i, 0)),
        pl.BlockSpec((tb, NUM_EXPERTS), lambda i: (i, 0)),
    )

    return pl.pallas_call(
        _gating_kernel,
        grid=grid,
        in_specs=in_specs,
        out_specs=out_specs,
        out_shape=out_shape,
        compiler_params=pltpu.CompilerParams(
            dimension_semantics=("parallel",)),
    )(*args)
